# baseline (speedup 1.0000x reference)
import functools
import jax
import jax.numpy as jnp
from jax import lax
from jax.experimental import pallas as pl
from jax.experimental.pallas import tpu as pltpu
from jax.experimental.pallas import tpu_sc as plsc

N = 10000
E = 320000
NC, NS, L = 2, 16, 16
NW = NC * NS
EPW = E // NW
HB = 640


HBL = HB * L
DW = 128


def _hist_body(dst_hbm, zeros_hbm, out_hbm, dst_v, ones_v, shared_ref):
    c = lax.axis_index("c")
    s = lax.axis_index("s")
    wid = c * NS + s

    ones16 = jnp.ones((L,), jnp.float32)
    for k in range(L):
        for j in range(DW // L):
            ones_v[k, pl.ds(j * L, L)] = ones16

    @pl.when(s == 0)
    def _():
        pltpu.sync_copy(zeros_hbm, shared_ref)

    pltpu.sync_copy(dst_hbm.at[pl.ds(wid * EPW, EPW)], dst_v)
    plsc.subcore_barrier()

    def body(i, _):
        idx = dst_v[pl.ds(i * L, L)]
        pltpu.sync_copy(ones_v, shared_ref.at[idx], add=True)
        return 0

    lax.fori_loop(0, EPW // L, body, 0)
    plsc.subcore_barrier()

    @pl.when(s == 0)
    def _():
        pltpu.sync_copy(shared_ref, out_hbm.at[pl.ds(c * HBL, HBL)])


@functools.partial(jax.jit, static_argnames=())
def _degree_hist(dst_i32):
    mesh = plsc.VectorSubcoreMesh(core_axis_name="c", subcore_axis_name="s")
    f = pl.kernel(
        _hist_body,
        out_type=jax.ShapeDtypeStruct((NC * HBL, DW), jnp.float32),
        mesh=mesh,
        scratch_types=[
            pltpu.VMEM((EPW,), jnp.int32),
            pltpu.VMEM((L, DW), jnp.float32),
            pltpu.VMEM_SHARED((HBL, DW), jnp.float32),
        ],
    )
    return f(dst_i32, jnp.zeros((HBL, DW), jnp.float32))


D2 = 128
NB = 5
CPT = EPW // L


def _seg_body(hs2_hbm, src_hbm, dst_hbm, zeros_hbm, out_hbm,
              src_v, dst_v, gsem, ssem, shared_ref, *rows_v):
    c = lax.axis_index("c")
    s = lax.axis_index("s")
    wid = c * NS + s

    @pl.when(jnp.logical_and(s == 0, c == 0))
    def _():
        pltpu.sync_copy(hs2_hbm, shared_ref)

    @pl.when(jnp.logical_and(s == 0, c == 1))
    def _():
        pltpu.sync_copy(zeros_hbm, shared_ref)

    pltpu.sync_copy(src_hbm.at[pl.ds(wid * EPW, EPW)], src_v)
    pltpu.sync_copy(dst_hbm.at[pl.ds(wid * EPW, EPW)], dst_v)
    plsc.subcore_barrier()

    def round_(o, _):
        descs = []
        for b in range(NB):
            idx_ref = src_v.at[pl.ds((o * NB + b) * L, L)]
            descs.append(pltpu.async_copy(hs2_hbm.at[idx_ref], rows_v[b], gsem))
        for d in descs:
            d.wait()
        descs = []
        for b in range(NB):
            idx = dst_v[pl.ds((o * NB + b) * L, L)]
            descs.append(pltpu.async_copy(rows_v[b], shared_ref.at[idx],
                                          ssem, add=True))
        for d in descs:
            d.wait()
        return 0

    lax.fori_loop(0, CPT // NB, round_, 0)
    plsc.subcore_barrier()

    @pl.when(s == 0)
    def _():
        pltpu.sync_copy(shared_ref, out_hbm.at[pl.ds(c * N, N)])


@functools.partial(jax.jit, static_argnames=())
def _seg_sum(hs2, src_i32, dst_i32):
    mesh = plsc.VectorSubcoreMesh(core_axis_name="c", subcore_axis_name="s")
    f = pl.kernel(
        _seg_body,
        out_type=jax.ShapeDtypeStruct((NC * N, D2), jnp.float32),
        mesh=mesh,
        scratch_types=[
            pltpu.VMEM((EPW,), jnp.int32),
            pltpu.VMEM((EPW,), jnp.int32),
            pltpu.SemaphoreType.DMA,
            pltpu.SemaphoreType.DMA,
            pltpu.VMEM_SHARED((N, D2), jnp.float32),
        ] + [pltpu.VMEM((L, D2), jnp.float32) for _ in range(NB)],
    )
    return f(hs2, src_i32, dst_i32, jnp.zeros((N, D2), jnp.float32))


BN = 1000
BT = 2480


def _hs2_body(counts_ref, x_ref, eh_ref, wn_ref, we_ref, hs2_ref, a_ref):
    a = lax.rsqrt(counts_ref[...] + 1.0)
    h = jnp.dot(x_ref[...], wn_ref[...], preferred_element_type=jnp.float32)
    he = jnp.dot(eh_ref[...], we_ref[...], preferred_element_type=jnp.float32)
    hs2_ref[...] = jnp.concatenate([h * a, he * a], axis=1)
    a_ref[...] = a


_hs2_call = pl.pallas_call(
    _hs2_body,
    grid=(N // BN,),
    in_specs=[
        pl.BlockSpec((BN, 1), lambda i: (i, 0)),
        pl.BlockSpec((BN, 128), lambda i: (i, 0)),
        pl.BlockSpec((BN, 16), lambda i: (i, 0)),
        pl.BlockSpec((128, 64), lambda i: (0, 0)),
        pl.BlockSpec((16, 64), lambda i: (0, 0)),
    ],
    out_specs=[pl.BlockSpec((BN, 128), lambda i: (i, 0)),
               pl.BlockSpec((BN, 1), lambda i: (i, 0))],
    out_shape=[jax.ShapeDtypeStruct((N, 128), jnp.float32),
               jax.ShapeDtypeStruct((N, 1), jnp.float32)],
)


def _tail_body(et_ref, we_ref, be_ref, out_ref):
    i = pl.program_id(0)
    r = jnp.maximum(jnp.dot(et_ref[...], we_ref[...],
                            preferred_element_type=jnp.float32) + be_ref[...],
                    0.0)
    col = jnp.sum(r, axis=0, keepdims=True)

    @pl.when(i == 0)
    def _():
        out_ref[...] = col

    @pl.when(i > 0)
    def _():
        out_ref[...] = out_ref[...] + col


_tail_call = pl.pallas_call(
    _tail_body,
    grid=((E - N) // BT,),
    in_specs=[
        pl.BlockSpec((BT, 16), lambda i: (i, 0)),
        pl.BlockSpec((16, 64), lambda i: (0, 0)),
        pl.BlockSpec((1, 64), lambda i: (0, 0)),
    ],
    out_specs=pl.BlockSpec((1, 64), lambda i: (0, 0)),
    out_shape=jax.ShapeDtypeStruct((1, 64), jnp.float32),
)


def _colsum_body(s0_ref, s1_ref, a_ref, bcat_ref, acc_ref):
    i = pl.program_id(0)
    blk = s0_ref[...] + s1_ref[...]
    out2 = jnp.maximum(a_ref[...] * blk + bcat_ref[...], 0.0)
    col = jnp.sum(out2, axis=0, keepdims=True)

    @pl.when(i == 0)
    def _():
        acc_ref[...] = col

    @pl.when(i > 0)
    def _():
        acc_ref[...] = acc_ref[...] + col


_colsum_call = pl.pallas_call(
    _colsum_body,
    grid=(N // BN,),
    in_specs=[
        pl.BlockSpec((BN, 128), lambda i: (i, 0)),
        pl.BlockSpec((BN, 128), lambda i: (i, 0)),
        pl.BlockSpec((BN, 1), lambda i: (i, 0)),
        pl.BlockSpec((1, 128), lambda i: (0, 0)),
    ],
    out_specs=pl.BlockSpec((1, 128), lambda i: (0, 0)),
    out_shape=jax.ShapeDtypeStruct((1, 128), jnp.float32),
)


def _fc_body(col_ref, tail_ref, scale_ref, wfc_ref, bfc_ref, out_ref):
    z = (col_ref[...] + tail_ref[...]) * scale_ref[...]
    logits = jnp.dot(z, wfc_ref[...], preferred_element_type=jnp.float32)
    logits = logits + bfc_ref[...]
    m = jnp.max(logits, axis=1, keepdims=True)
    p = jnp.exp(logits - m)
    out_ref[...] = p / jnp.sum(p, axis=1, keepdims=True)


_fc_call = pl.pallas_call(
    _fc_body,
    out_shape=jax.ShapeDtypeStruct((1, 128), jnp.float32),
)


def kernel(x, edge_index, edge_attr, W_node, b_node, W_edge, b_edge, W_fc, b_fc):
    src = edge_index[0].astype(jnp.int32)
    dst = edge_index[1].astype(jnp.int32)

    hist2 = _degree_hist(dst)
    counts = (hist2[:HBL, 0:1] + hist2[HBL:, 0:1])[:N]

    hs2, a2 = _hs2_call(counts, x, edge_attr[:N], W_node, W_edge)
    tail = _tail_call(edge_attr[N:], W_edge, b_edge[None, :])

    s2p = _seg_sum(hs2, src, dst)
    bcat = jnp.concatenate([b_node, b_edge])[None, :]
    col = _colsum_call(s2p[:N], s2p[N:], a2, bcat)

    scale = jnp.concatenate([jnp.full((64,), 1.0 / N, jnp.float32),
                             jnp.full((64,), 1.0 / E, jnp.float32)])[None, :]
    tail_pad = jnp.concatenate([jnp.zeros((1, 64), jnp.float32), tail], axis=1)
    wfc_pad = jnp.pad(W_fc, ((0, 0), (0, 128 - W_fc.shape[1])))
    bfc_pad = jnp.concatenate(
        [b_fc[None, :], jnp.full((1, 128 - b_fc.shape[0]), -1e30, jnp.float32)],
        axis=1)
    out = _fc_call(col, tail_pad, scale, wfc_pad, bfc_pad)
    return out[:, :6]


# speedup vs baseline: 1.0726x; 1.0726x over previous
import functools
import jax
import jax.numpy as jnp
from jax import lax
from jax.experimental import pallas as pl
from jax.experimental.pallas import tpu as pltpu
from jax.experimental.pallas import tpu_sc as plsc

N = 10000
E = 320000
NC, NS, L = 2, 16, 16
NW = NC * NS
EPW = E // NW
HB = 640


HBL = HB * L
DW = 128


def _hist_body(dst_hbm, zeros_hbm, out_hbm, dst_v, ones_v, shared_ref, hsem):
    c = lax.axis_index("c")
    s = lax.axis_index("s")
    wid = c * NS + s

    ones16 = jnp.ones((L,), jnp.float32)
    for k in range(L):
        for j in range(DW // L):
            ones_v[k, pl.ds(j * L, L)] = ones16

    @pl.when(s == 0)
    def _():
        pltpu.sync_copy(zeros_hbm, shared_ref)

    pltpu.sync_copy(dst_hbm.at[pl.ds(wid * EPW, EPW)], dst_v)
    plsc.subcore_barrier()

    def body(o, _):
        descs = []
        for b in range(5):
            idx = dst_v[pl.ds((o * 5 + b) * L, L)]
            descs.append(pltpu.async_copy(ones_v, shared_ref.at[idx],
                                          hsem, add=True))
        for d in descs:
            d.wait()
        return 0

    lax.fori_loop(0, EPW // L // 5, body, 0)
    plsc.subcore_barrier()

    @pl.when(s == 0)
    def _():
        pltpu.sync_copy(shared_ref, out_hbm.at[pl.ds(c * HBL, HBL)])


@functools.partial(jax.jit, static_argnames=())
def _degree_hist(dst_i32):
    mesh = plsc.VectorSubcoreMesh(core_axis_name="c", subcore_axis_name="s")
    f = pl.kernel(
        _hist_body,
        out_type=jax.ShapeDtypeStruct((NC * HBL, DW), jnp.float32),
        mesh=mesh,
        scratch_types=[
            pltpu.VMEM((EPW,), jnp.int32),
            pltpu.VMEM((L, DW), jnp.float32),
            pltpu.VMEM_SHARED((HBL, DW), jnp.float32),
            pltpu.SemaphoreType.DMA,
        ],
    )
    return f(dst_i32, jnp.zeros((HBL, DW), jnp.float32))


D2 = 128
NB = 5
CPT = EPW // L


def _seg_body(hs2_hbm, src_hbm, dst_hbm, zeros_hbm, out_hbm,
              src_v, dst_v, gsem, ssem, shared_ref, *rows_v):
    c = lax.axis_index("c")
    s = lax.axis_index("s")
    wid = c * NS + s

    @pl.when(jnp.logical_and(s == 0, c == 0))
    def _():
        pltpu.sync_copy(hs2_hbm, shared_ref)

    @pl.when(jnp.logical_and(s == 0, c == 1))
    def _():
        pltpu.sync_copy(zeros_hbm, shared_ref)

    pltpu.sync_copy(src_hbm.at[pl.ds(wid * EPW, EPW)], src_v)
    pltpu.sync_copy(dst_hbm.at[pl.ds(wid * EPW, EPW)], dst_v)
    plsc.subcore_barrier()

    def round_(o, _):
        descs = []
        for b in range(NB):
            idx_ref = src_v.at[pl.ds((o * NB + b) * L, L)]
            descs.append(pltpu.async_copy(hs2_hbm.at[idx_ref], rows_v[b], gsem))
        for d in descs:
            d.wait()
        descs = []
        for b in range(NB):
            idx = dst_v[pl.ds((o * NB + b) * L, L)]
            descs.append(pltpu.async_copy(rows_v[b], shared_ref.at[idx],
                                          ssem, add=True))
        for d in descs:
            d.wait()
        return 0

    lax.fori_loop(0, CPT // NB, round_, 0)
    plsc.subcore_barrier()

    @pl.when(s == 0)
    def _():
        pltpu.sync_copy(shared_ref, out_hbm.at[pl.ds(c * N, N)])


@functools.partial(jax.jit, static_argnames=())
def _seg_sum(hs2, src_i32, dst_i32):
    mesh = plsc.VectorSubcoreMesh(core_axis_name="c", subcore_axis_name="s")
    f = pl.kernel(
        _seg_body,
        out_type=jax.ShapeDtypeStruct((NC * N, D2), jnp.float32),
        mesh=mesh,
        scratch_types=[
            pltpu.VMEM((EPW,), jnp.int32),
            pltpu.VMEM((EPW,), jnp.int32),
            pltpu.SemaphoreType.DMA,
            pltpu.SemaphoreType.DMA,
            pltpu.VMEM_SHARED((N, D2), jnp.float32),
        ] + [pltpu.VMEM((L, D2), jnp.float32) for _ in range(NB)],
    )
    return f(hs2, src_i32, dst_i32, jnp.zeros((N, D2), jnp.float32))


BN = 1000
BT = 2480


def _hs2_body(counts_ref, x_ref, eh_ref, wn_ref, we_ref, hs2_ref, a_ref):
    a = lax.rsqrt(counts_ref[...] + 1.0)
    h = jnp.dot(x_ref[...], wn_ref[...], preferred_element_type=jnp.float32)
    he = jnp.dot(eh_ref[...], we_ref[...], preferred_element_type=jnp.float32)
    hs2_ref[...] = jnp.concatenate([h * a, he * a], axis=1)
    a_ref[...] = a


_hs2_call = pl.pallas_call(
    _hs2_body,
    grid=(N // BN,),
    in_specs=[
        pl.BlockSpec((BN, 1), lambda i: (i, 0)),
        pl.BlockSpec((BN, 128), lambda i: (i, 0)),
        pl.BlockSpec((BN, 16), lambda i: (i, 0)),
        pl.BlockSpec((128, 64), lambda i: (0, 0)),
        pl.BlockSpec((16, 64), lambda i: (0, 0)),
    ],
    out_specs=[pl.BlockSpec((BN, 128), lambda i: (i, 0)),
               pl.BlockSpec((BN, 1), lambda i: (i, 0))],
    out_shape=[jax.ShapeDtypeStruct((N, 128), jnp.float32),
               jax.ShapeDtypeStruct((N, 1), jnp.float32)],
)


def _tail_body(et_ref, we_ref, be_ref, out_ref):
    i = pl.program_id(0)
    r = jnp.maximum(jnp.dot(et_ref[...], we_ref[...],
                            preferred_element_type=jnp.float32) + be_ref[...],
                    0.0)
    col = jnp.sum(r, axis=0, keepdims=True)

    @pl.when(i == 0)
    def _():
        out_ref[...] = col

    @pl.when(i > 0)
    def _():
        out_ref[...] = out_ref[...] + col


_tail_call = pl.pallas_call(
    _tail_body,
    grid=((E - N) // BT,),
    in_specs=[
        pl.BlockSpec((BT, 16), lambda i: (i, 0)),
        pl.BlockSpec((16, 64), lambda i: (0, 0)),
        pl.BlockSpec((1, 64), lambda i: (0, 0)),
    ],
    out_specs=pl.BlockSpec((1, 64), lambda i: (0, 0)),
    out_shape=jax.ShapeDtypeStruct((1, 64), jnp.float32),
)


def _colsum_body(s0_ref, s1_ref, a_ref, bcat_ref, acc_ref):
    i = pl.program_id(0)
    blk = s0_ref[...] + s1_ref[...]
    out2 = jnp.maximum(a_ref[...] * blk + bcat_ref[...], 0.0)
    col = jnp.sum(out2, axis=0, keepdims=True)

    @pl.when(i == 0)
    def _():
        acc_ref[...] = col

    @pl.when(i > 0)
    def _():
        acc_ref[...] = acc_ref[...] + col


_colsum_call = pl.pallas_call(
    _colsum_body,
    grid=(N // BN,),
    in_specs=[
        pl.BlockSpec((BN, 128), lambda i: (i, 0)),
        pl.BlockSpec((BN, 128), lambda i: (i, 0)),
        pl.BlockSpec((BN, 1), lambda i: (i, 0)),
        pl.BlockSpec((1, 128), lambda i: (0, 0)),
    ],
    out_specs=pl.BlockSpec((1, 128), lambda i: (0, 0)),
    out_shape=jax.ShapeDtypeStruct((1, 128), jnp.float32),
)


def _fc_body(col_ref, tail_ref, scale_ref, wfc_ref, bfc_ref, out_ref):
    z = (col_ref[...] + tail_ref[...]) * scale_ref[...]
    logits = jnp.dot(z, wfc_ref[...], preferred_element_type=jnp.float32)
    logits = logits + bfc_ref[...]
    m = jnp.max(logits, axis=1, keepdims=True)
    p = jnp.exp(logits - m)
    out_ref[...] = p / jnp.sum(p, axis=1, keepdims=True)


_fc_call = pl.pallas_call(
    _fc_body,
    out_shape=jax.ShapeDtypeStruct((1, 128), jnp.float32),
)


def kernel(x, edge_index, edge_attr, W_node, b_node, W_edge, b_edge, W_fc, b_fc):
    src = edge_index[0].astype(jnp.int32)
    dst = edge_index[1].astype(jnp.int32)

    hist2 = _degree_hist(dst)
    counts = (hist2[:HBL, 0:1] + hist2[HBL:, 0:1])[:N]

    hs2, a2 = _hs2_call(counts, x, edge_attr[:N], W_node, W_edge)
    tail = _tail_call(edge_attr[N:], W_edge, b_edge[None, :])

    s2p = _seg_sum(hs2, src, dst)
    bcat = jnp.concatenate([b_node, b_edge])[None, :]
    col = _colsum_call(s2p[:N], s2p[N:], a2, bcat)

    scale = jnp.concatenate([jnp.full((64,), 1.0 / N, jnp.float32),
                             jnp.full((64,), 1.0 / E, jnp.float32)])[None, :]
    tail_pad = jnp.concatenate([jnp.zeros((1, 64), jnp.float32), tail], axis=1)
    wfc_pad = jnp.pad(W_fc, ((0, 0), (0, 128 - W_fc.shape[1])))
    bfc_pad = jnp.concatenate(
        [b_fc[None, :], jnp.full((1, 128 - b_fc.shape[0]), -1e30, jnp.float32)],
        axis=1)
    out = _fc_call(col, tail_pad, scale, wfc_pad, bfc_pad)
    return out[:, :6]
